# baseline (speedup 1.0000x reference)
import functools

import jax
import jax.numpy as jnp
from jax import lax
from jax.experimental import pallas as pl
from jax.experimental.pallas import tpu as pltpu
from jax.experimental.pallas import tpu_sc as plsc

MAXZ = 54
NODE = 128
NW = 32
CHUNK = 128
CHUNKS_PER_W = 25
PER_W = CHUNK * CHUNKS_PER_W
B_PAD = NW * PER_W

NSLOT = 4
GRP = CHUNK // 16

_mesh = plsc.VectorSubcoreMesh(core_axis_name="c", subcore_axis_name="s")


@functools.partial(
    pl.kernel,
    mesh=_mesh,
    out_type=jax.ShapeDtypeStruct((B_PAD * NODE,), jnp.float32),
    scratch_types=[
        pltpu.VMEM((MAXZ * NODE,), jnp.float32),
        pltpu.VMEM((PER_W,), jnp.int32),
        pltpu.VMEM((NSLOT * CHUNK * NODE,), jnp.float32),
        pltpu.SemaphoreType.DMA((NSLOT,)),
    ],
    compiler_params=pltpu.CompilerParams(needs_layout_passes=False),
)
def _embed_lookup(table_hbm, z_hbm, out_hbm, table_v, idx_v, bufs, ssem):
    wid = lax.axis_index("s") * 2 + lax.axis_index("c")
    pltpu.sync_copy(table_hbm, table_v)
    pltpu.sync_copy(z_hbm.at[pl.ds(wid * PER_W, PER_W)], idx_v)

    lanes = lax.iota(jnp.int32, 16)
    row_flat = [(lanes + 16 * g) * NODE for g in range(GRP)]

    scatters = {}
    for i in range(CHUNKS_PER_W):
        b = i % NSLOT
        if i >= NSLOT:
            scatters[i - NSLOT].wait()
        buf = bufs.at[pl.ds(b * CHUNK * NODE, CHUNK * NODE)]
        zb = [idx_v[pl.ds(i * CHUNK + 16 * g, 16)] * NODE for g in range(GRP)]

        @plsc.parallel_loop(0, NODE, unroll=2, carry=lax.iota(jnp.int32, 16))
        def _cols(c, cvec):
            for g in range(GRP):
                vals = plsc.load_gather(table_v, [zb[g] + cvec])
                plsc.store_scatter(buf, [row_flat[g] + cvec], vals)
            return (cvec + 1) & (NODE - 1)

        off = (wid * CHUNKS_PER_W + i) * (CHUNK * NODE)
        scatters[i] = pltpu.async_copy(
            buf, out_hbm.at[pl.ds(off, CHUNK * NODE)], ssem.at[b]
        )

    for i in range(CHUNKS_PER_W - NSLOT, CHUNKS_PER_W):
        scatters[i].wait()


def kernel(Z, table):
    z_pad = jnp.pad(Z.astype(jnp.int32), (0, B_PAD - Z.shape[0]))
    out = _embed_lookup(table.reshape(-1), z_pad)
    return out.reshape(B_PAD, NODE)[: Z.shape[0]]


# speedup vs baseline: 1.5939x; 1.5939x over previous
import functools

import jax
import jax.numpy as jnp
from jax import lax
from jax.experimental import pallas as pl
from jax.experimental.pallas import tpu as pltpu
from jax.experimental.pallas import tpu_sc as plsc

MAXZ = 54
NODE = 128
NW = 32
CHUNK = 128
CHUNKS_PER_W = 25
PER_W = CHUNK * CHUNKS_PER_W
B_PAD = NW * PER_W
N_OUT = 100000
REM = N_OUT % CHUNK

NSLOT = 4
GRP = CHUNK // 16
CN = CHUNK * NODE

_mesh = plsc.VectorSubcoreMesh(core_axis_name="c", subcore_axis_name="s")


@functools.partial(
    pl.kernel,
    mesh=_mesh,
    out_type=jax.ShapeDtypeStruct((N_OUT * NODE,), jnp.float32),
    scratch_types=[
        pltpu.VMEM((MAXZ * NODE,), jnp.float32),
        pltpu.VMEM((PER_W,), jnp.int32),
        pltpu.VMEM((NSLOT * CN,), jnp.float32),
        pltpu.SemaphoreType.DMA((NSLOT,)),
    ],
    compiler_params=pltpu.CompilerParams(needs_layout_passes=False),
)
def _embed_lookup(table_hbm, z_hbm, out_hbm, table_v, idx_v, bufs, ssem):
    wid = lax.axis_index("s") * 2 + lax.axis_index("c")
    pltpu.sync_copy(table_hbm, table_v)
    pltpu.sync_copy(z_hbm.at[pl.ds(wid * PER_W, PER_W)], idx_v)

    lanes = lax.iota(jnp.int32, 16)
    row_flat = [(lanes + 16 * g) * NODE for g in range(GRP)]

    def mk_copies(i, b):
        row_base = wid * PER_W + i * CHUNK
        off = row_base * NODE
        full = pltpu.make_async_copy(
            bufs.at[pl.ds(b * CN, CN)], out_hbm.at[pl.ds(off, CN)], ssem.at[b]
        )
        part = pltpu.make_async_copy(
            bufs.at[pl.ds(b * CN, REM * NODE)],
            out_hbm.at[pl.ds(off, REM * NODE)],
            ssem.at[b],
        )
        is_full = row_base + CHUNK <= N_OUT
        is_part = jnp.logical_and(row_base < N_OUT, row_base + CHUNK > N_OUT)
        return full, part, is_full, is_part

    for i in range(CHUNKS_PER_W):
        b = i % NSLOT
        if i >= NSLOT:
            pf, pp, pif, pip = mk_copies(i - NSLOT, b)
            pl.when(pif)(pf.wait)
            pl.when(pip)(pp.wait)
        buf = bufs.at[pl.ds(b * CN, CN)]
        zb = [idx_v[pl.ds(i * CHUNK + 16 * g, 16)] * NODE for g in range(GRP)]

        @plsc.parallel_loop(0, NODE, unroll=2, carry=lax.iota(jnp.int32, 16))
        def _cols(c, cvec):
            for g in range(GRP):
                vals = plsc.load_gather(table_v, [zb[g] + cvec])
                plsc.store_scatter(buf, [row_flat[g] + cvec], vals)
            return (cvec + 1) & (NODE - 1)

        full, part, is_full, is_part = mk_copies(i, b)
        pl.when(is_full)(full.start)
        pl.when(is_part)(part.start)

    for i in range(CHUNKS_PER_W - NSLOT, CHUNKS_PER_W):
        full, part, is_full, is_part = mk_copies(i, i % NSLOT)
        pl.when(is_full)(full.wait)
        pl.when(is_part)(part.wait)


def kernel(Z, table):
    z_pad = jnp.pad(Z.astype(jnp.int32), (0, B_PAD - Z.shape[0]))
    out = _embed_lookup(table.reshape(-1), z_pad)
    return out.reshape(N_OUT, NODE)


# speedup vs baseline: 1.6179x; 1.0151x over previous
import functools

import jax
import jax.numpy as jnp
from jax import lax
from jax.experimental import pallas as pl
from jax.experimental.pallas import tpu as pltpu
from jax.experimental.pallas import tpu_sc as plsc

MAXZ = 54
NODE = 128
NW = 32
CHUNK = 128
CHUNKS_PER_W = 25
PER_W = CHUNK * CHUNKS_PER_W
N_OUT = 100000

NSLOT = 4
GRP = CHUNK // 16
CN = CHUNK * NODE

_mesh = plsc.VectorSubcoreMesh(core_axis_name="c", subcore_axis_name="s")


@functools.partial(
    pl.kernel,
    mesh=_mesh,
    out_type=jax.ShapeDtypeStruct((N_OUT * NODE,), jnp.float32),
    scratch_types=[
        pltpu.VMEM((MAXZ * NODE,), jnp.float32),
        pltpu.VMEM((PER_W,), jnp.int32),
        pltpu.VMEM((NSLOT * CN,), jnp.float32),
        pltpu.SemaphoreType.DMA((NSLOT,)),
    ],
    compiler_params=pltpu.CompilerParams(needs_layout_passes=False),
)
def _embed_lookup(table_hbm, z_hbm, out_hbm, table_v, idx_v, bufs, ssem):
    wid = lax.axis_index("s") * 2 + lax.axis_index("c")
    base = jnp.minimum(wid * PER_W, N_OUT - PER_W)
    pltpu.sync_copy(table_hbm, table_v)
    pltpu.sync_copy(z_hbm.at[pl.ds(base, PER_W)], idx_v)

    lanes = lax.iota(jnp.int32, 16)
    row_flat = [(lanes + 16 * g) * NODE for g in range(GRP)]

    scatters = {}
    for i in range(CHUNKS_PER_W):
        b = i % NSLOT
        if i >= NSLOT:
            scatters[i - NSLOT].wait()
        buf = bufs.at[pl.ds(b * CN, CN)]
        zb = [idx_v[pl.ds(i * CHUNK + 16 * g, 16)] * NODE for g in range(GRP)]

        @plsc.parallel_loop(0, NODE, unroll=2, carry=lax.iota(jnp.int32, 16))
        def _cols(c, cvec):
            for g in range(GRP):
                vals = plsc.load_gather(table_v, [zb[g] + cvec])
                plsc.store_scatter(buf, [row_flat[g] + cvec], vals)
            return (cvec + 1) & (NODE - 1)

        off = (base + i * CHUNK) * NODE
        scatters[i] = pltpu.async_copy(
            buf, out_hbm.at[pl.ds(off, CN)], ssem.at[b]
        )

    for i in range(CHUNKS_PER_W - NSLOT, CHUNKS_PER_W):
        scatters[i].wait()


def kernel(Z, table):
    out = _embed_lookup(table.reshape(-1), Z.astype(jnp.int32))
    return out.reshape(N_OUT, NODE)


# speedup vs baseline: 1.8426x; 1.1389x over previous
import functools

import jax
import jax.numpy as jnp
from jax import lax
from jax.experimental import pallas as pl
from jax.experimental.pallas import tpu as pltpu
from jax.experimental.pallas import tpu_sc as plsc

MAXZ = 54
NODE = 128
NW = 32
CHUNK = 128
CHUNKS_PER_W = 25
PER_W = CHUNK * CHUNKS_PER_W
N_OUT = 100000

NSLOT = 4
GRP = CHUNK // 16
CN = CHUNK * NODE

_mesh = plsc.VectorSubcoreMesh(core_axis_name="c", subcore_axis_name="s")


@functools.partial(
    pl.kernel,
    mesh=_mesh,
    out_type=jax.ShapeDtypeStruct((N_OUT * NODE,), jnp.float32),
    scratch_types=[
        pltpu.VMEM((MAXZ * NODE,), jnp.float32),
        pltpu.VMEM((PER_W,), jnp.int32),
        pltpu.VMEM((NSLOT * CN,), jnp.float32),
        pltpu.SemaphoreType.DMA((NSLOT,)),
    ],
    compiler_params=pltpu.CompilerParams(needs_layout_passes=False),
)
def _embed_lookup(table_hbm, z_hbm, out_hbm, table_v, idx_v, bufs, ssem):
    wid = lax.axis_index("s") * 2 + lax.axis_index("c")
    base = jnp.minimum(wid * PER_W, N_OUT - PER_W)
    pltpu.sync_copy(table_hbm, table_v)
    pltpu.sync_copy(z_hbm.at[pl.ds(base, PER_W)], idx_v)

    lanes = lax.iota(jnp.int32, 16)
    row_flat = [(lanes + 16 * g) * NODE for g in range(GRP)]

    def chunk_copy(i, b):
        off = (base + i * CHUNK) * NODE
        return pltpu.make_async_copy(
            bufs.at[pl.ds(b * CN, CN)], out_hbm.at[pl.ds(off, CN)], ssem.at[b]
        )

    def body(i, carry):
        b = lax.rem(i, NSLOT)

        @pl.when(i >= NSLOT)
        def _drain():
            chunk_copy(i - NSLOT, b).wait()

        buf = bufs.at[pl.ds(b * CN, CN)]
        zb = [idx_v[pl.ds(i * CHUNK + 16 * g, 16)] * NODE for g in range(GRP)]

        @plsc.parallel_loop(0, NODE, unroll=2, carry=lax.iota(jnp.int32, 16))
        def _cols(c, cvec):
            for g in range(GRP):
                vals = plsc.load_gather(table_v, [zb[g] + cvec])
                plsc.store_scatter(buf, [row_flat[g] + cvec], vals)
            return (cvec + 1) & (NODE - 1)

        chunk_copy(i, b).start()
        return carry

    lax.fori_loop(0, CHUNKS_PER_W, body, 0)

    def drain(i, carry):
        chunk_copy(i, lax.rem(i, NSLOT)).wait()
        return carry

    lax.fori_loop(CHUNKS_PER_W - NSLOT, CHUNKS_PER_W, drain, 0)


def kernel(Z, table):
    out = _embed_lookup(table.reshape(-1), Z.astype(jnp.int32))
    return out.reshape(N_OUT, NODE)


# speedup vs baseline: 1.8695x; 1.0146x over previous
import functools

import jax
import jax.numpy as jnp
from jax import lax
from jax.experimental import pallas as pl
from jax.experimental.pallas import tpu as pltpu
from jax.experimental.pallas import tpu_sc as plsc

MAXZ = 54
NODE = 128
NW = 32
CHUNK = 128
CHUNKS_PER_W = 25
PER_W = CHUNK * CHUNKS_PER_W
N_OUT = 100000

NSLOT = 4
GRP = CHUNK // 16
CN = CHUNK * NODE

_mesh = plsc.VectorSubcoreMesh(core_axis_name="c", subcore_axis_name="s")


@functools.partial(
    pl.kernel,
    mesh=_mesh,
    out_type=jax.ShapeDtypeStruct((N_OUT * NODE,), jnp.float32),
    scratch_types=[
        pltpu.VMEM((MAXZ * NODE,), jnp.float32),
        pltpu.VMEM((PER_W,), jnp.int32),
        pltpu.VMEM((NSLOT * CN,), jnp.float32),
        pltpu.SemaphoreType.DMA((NSLOT,)),
        pltpu.SemaphoreType.DMA((2,)),
    ],
    compiler_params=pltpu.CompilerParams(needs_layout_passes=False),
)
def _embed_lookup(table_hbm, z_hbm, out_hbm, table_v, idx_v, bufs, ssem, lsem):
    wid = lax.axis_index("s") * 2 + lax.axis_index("c")
    base = jnp.minimum(wid * PER_W, N_OUT - PER_W)
    tcp = pltpu.async_copy(table_hbm, table_v, lsem.at[0])
    icp = pltpu.async_copy(z_hbm.at[pl.ds(base, PER_W)], idx_v, lsem.at[1])
    tcp.wait()
    icp.wait()

    lanes = lax.iota(jnp.int32, 16)
    row_flat = [(lanes + 16 * g) * NODE for g in range(GRP)]

    def chunk_copy(i, b):
        off = (base + i * CHUNK) * NODE
        return pltpu.make_async_copy(
            bufs.at[pl.ds(b * CN, CN)], out_hbm.at[pl.ds(off, CN)], ssem.at[b]
        )

    def body(i, carry):
        b = lax.rem(i, NSLOT)

        @pl.when(i >= NSLOT)
        def _drain():
            chunk_copy(i - NSLOT, b).wait()

        buf = bufs.at[pl.ds(b * CN, CN)]
        zb = [idx_v[pl.ds(i * CHUNK + 16 * g, 16)] * NODE for g in range(GRP)]

        @plsc.parallel_loop(0, NODE, unroll=2, carry=lax.iota(jnp.int32, 16))
        def _cols(c, cvec):
            for g in range(GRP):
                vals = plsc.load_gather(table_v, [zb[g] + cvec])
                plsc.store_scatter(buf, [row_flat[g] + cvec], vals)
            return (cvec + 1) & (NODE - 1)

        chunk_copy(i, b).start()
        return carry

    lax.fori_loop(0, CHUNKS_PER_W, body, 0)

    def drain(i, carry):
        chunk_copy(i, lax.rem(i, NSLOT)).wait()
        return carry

    lax.fori_loop(CHUNKS_PER_W - NSLOT, CHUNKS_PER_W, drain, 0)


def kernel(Z, table):
    out = _embed_lookup(table.reshape(-1), Z.astype(jnp.int32))
    return out.reshape(N_OUT, NODE)
